# baseline (speedup 1.0000x reference)
import jax
import jax.numpy as jnp
from jax.experimental import pallas as pl
from jax.experimental.pallas import tpu as pltpu

_B, _N, _D = 4, 4096, 4096
_E = 64
_K = 8
_T = 512


def _router_kernel(x_ref, w_ref, wts_ref, idx_ref, part_ref):
    logits = jnp.dot(x_ref[...], w_ref[...], preferred_element_type=jnp.float32)
    m = jnp.max(logits, axis=-1, keepdims=True)
    p = jnp.exp(logits - m)
    s = jnp.sum(p, axis=-1, keepdims=True)

    part_ref[0, ...] = jnp.sum(p * (1.0 / s), axis=0, keepdims=True)

    iota = jax.lax.broadcasted_iota(jnp.int32, p.shape, 1)
    packed = (jax.lax.bitcast_convert_type(p, jnp.int32) & ~(_E - 1)) | (
        (_E - 1) - iota
    )
    cur = packed
    vals, idxs = [], []
    for _ in range(_K):
        mk = jnp.max(cur, axis=-1, keepdims=True)
        idxs.append((_E - 1) - (mk & (_E - 1)))
        vals.append(jax.lax.bitcast_convert_type(mk & ~(_E - 1), jnp.float32))
        cur = jnp.where(cur == mk, jnp.int32(-(2**31)), cur)
    v = jnp.concatenate(vals, axis=-1)
    wts_ref[...] = v / jnp.sum(v, axis=-1, keepdims=True)
    idx_ref[...] = jnp.concatenate(idxs, axis=-1)


def _loss_kernel(part_ref, loss_ref):
    usage = jnp.sum(part_ref[...], axis=(0, 1)) * (1.0 / (_B * _N))
    loss_ref[0, 0] = jnp.sum(usage * jnp.log(usage * _E + 1e-8))


def kernel(x, gate_w):
    tokens = _B * _N
    x2 = x.reshape(tokens, _D)
    w = gate_w.T
    grid = tokens // _T
    wts, idx, part = pl.pallas_call(
        _router_kernel,
        grid=(grid,),
        in_specs=[
            pl.BlockSpec((_T, _D), lambda i: (i, 0)),
            pl.BlockSpec((_D, _E), lambda i: (0, 0)),
        ],
        out_specs=[
            pl.BlockSpec((_T, _K), lambda i: (i, 0)),
            pl.BlockSpec((_T, _K), lambda i: (i, 0)),
            pl.BlockSpec((1, 1, _E), lambda i: (i, 0, 0)),
        ],
        out_shape=[
            jax.ShapeDtypeStruct((tokens, _K), jnp.float32),
            jax.ShapeDtypeStruct((tokens, _K), jnp.int32),
            jax.ShapeDtypeStruct((grid, 1, _E), jnp.float32),
        ],
        compiler_params=pltpu.CompilerParams(
            dimension_semantics=("parallel",),
        ),
    )(x2, w)
    loss = pl.pallas_call(
        _loss_kernel,
        out_specs=pl.BlockSpec(memory_space=pltpu.SMEM),
        out_shape=jax.ShapeDtypeStruct((1, 1), jnp.float32),
    )(part)
    return (
        wts.reshape(_B, _N, _K),
        idx.reshape(_B, _N, _K),
        loss[0, 0],
    )


# speedup vs baseline: 1.1141x; 1.1141x over previous
import jax
import jax.numpy as jnp
from jax.experimental import pallas as pl
from jax.experimental.pallas import tpu as pltpu

_B, _N, _D = 4, 4096, 4096
_E = 64
_K = 8
_T = 1024


def _router_kernel(x_ref, w_ref, wts_ref, idx_ref, loss_ref, acc_ref):
    i = pl.program_id(0)
    nblocks = pl.num_programs(0)

    @pl.when(i == 0)
    def _init():
        acc_ref[...] = jnp.zeros_like(acc_ref)

    logits = jnp.dot(x_ref[...], w_ref[...], preferred_element_type=jnp.float32)
    m = jnp.max(logits, axis=-1, keepdims=True)
    p = jnp.exp(logits - m)
    s = jnp.sum(p, axis=-1, keepdims=True)

    acc_ref[...] += jnp.sum(p * (1.0 / s), axis=0, keepdims=True)

    iota = jax.lax.broadcasted_iota(jnp.int32, p.shape, 1)
    packed = ((jax.lax.bitcast_convert_type(p, jnp.int32) + 32) & ~(_E - 1)) | (
        (_E - 1) - iota
    )
    cur = packed
    vals, idxs = [], []
    for _ in range(_K):
        mk = jnp.max(cur, axis=-1, keepdims=True)
        idxs.append((_E - 1) - (mk & (_E - 1)))
        vals.append(jax.lax.bitcast_convert_type(mk & ~(_E - 1), jnp.float32))
        cur = jnp.where(cur == mk, jnp.int32(-(2**31)), cur)
    v = jnp.concatenate(vals, axis=-1)
    wts_ref[...] = v / jnp.sum(v, axis=-1, keepdims=True)
    idx_ref[...] = jnp.concatenate(idxs, axis=-1)

    @pl.when(i == nblocks - 1)
    def _finish():
        usage = acc_ref[...] / (nblocks * _T)
        loss_ref[0, 0] = jnp.sum(usage * jnp.log(usage * _E + 1e-8))


def kernel(x, gate_w):
    tokens = _B * _N
    x2 = x.reshape(tokens, _D)
    w = gate_w.T
    grid = tokens // _T
    wts, idx, loss = pl.pallas_call(
        _router_kernel,
        grid=(grid,),
        in_specs=[
            pl.BlockSpec((_T, _D), lambda i: (i, 0)),
            pl.BlockSpec((_D, _E), lambda i: (0, 0)),
        ],
        out_specs=[
            pl.BlockSpec((_T, _K), lambda i: (i, 0)),
            pl.BlockSpec((_T, _K), lambda i: (i, 0)),
            pl.BlockSpec(memory_space=pltpu.SMEM),
        ],
        out_shape=[
            jax.ShapeDtypeStruct((tokens, _K), jnp.float32),
            jax.ShapeDtypeStruct((tokens, _K), jnp.int32),
            jax.ShapeDtypeStruct((1, 1), jnp.float32),
        ],
        scratch_shapes=[pltpu.VMEM((1, _E), jnp.float32)],
        compiler_params=pltpu.CompilerParams(
            dimension_semantics=("arbitrary",),
        ),
    )(x2, w)
    return (
        wts.reshape(_B, _N, _K),
        idx.reshape(_B, _N, _K),
        loss[0, 0],
    )


# speedup vs baseline: 1.1246x; 1.0094x over previous
import jax
import jax.numpy as jnp
from jax.experimental import pallas as pl
from jax.experimental.pallas import tpu as pltpu

_B, _N, _D = 4, 4096, 4096
_E = 64
_K = 8
_T = 1024


def _router_kernel(xa_ref, xb_ref, wa_ref, wb_ref, wts_ref, idx_ref, loss_ref,
                   acc_ref):
    i = pl.program_id(0)
    nblocks = pl.num_programs(0)

    @pl.when(i == 0)
    def _init():
        acc_ref[...] = jnp.zeros_like(acc_ref)

    logits = jnp.dot(
        xa_ref[...], wa_ref[...], preferred_element_type=jnp.float32
    ) + jnp.dot(xb_ref[...], wb_ref[...], preferred_element_type=jnp.float32)
    m = jnp.max(logits, axis=-1, keepdims=True)
    p = jnp.exp(logits - m)
    s = jnp.sum(p, axis=-1, keepdims=True)

    acc_ref[...] += jnp.sum(p * (1.0 / s), axis=0, keepdims=True)

    iota = jax.lax.broadcasted_iota(jnp.int32, p.shape, 1)
    packed = ((jax.lax.bitcast_convert_type(p, jnp.int32) + 32) & ~(_E - 1)) | (
        (_E - 1) - iota
    )
    cur = packed
    vals, idxs = [], []
    for _ in range(_K):
        mk = jnp.max(cur, axis=-1, keepdims=True)
        idxs.append((_E - 1) - (mk & (_E - 1)))
        vals.append(jax.lax.bitcast_convert_type(mk & ~(_E - 1), jnp.float32))
        cur = jnp.where(cur == mk, jnp.int32(-(2**31)), cur)
    v = jnp.concatenate(vals, axis=-1)
    wts_ref[...] = v / jnp.sum(v, axis=-1, keepdims=True)
    idx_ref[...] = jnp.concatenate(idxs, axis=-1)

    @pl.when(i == nblocks - 1)
    def _finish():
        usage = acc_ref[...] / (nblocks * _T)
        loss_ref[0, 0] = jnp.sum(usage * jnp.log(usage * _E + 1e-8))


def kernel(x, gate_w):
    tokens = _B * _N
    x2 = x.reshape(tokens, _D)
    w = gate_w.T
    grid = tokens // _T
    wts, idx, loss = pl.pallas_call(
        _router_kernel,
        grid=(grid,),
        in_specs=[
            pl.BlockSpec((_T, _D // 2), lambda i: (i, 0)),
            pl.BlockSpec((_T, _D // 2), lambda i: (i, 1)),
            pl.BlockSpec((_D // 2, _E), lambda i: (0, 0)),
            pl.BlockSpec((_D // 2, _E), lambda i: (1, 0)),
        ],
        out_specs=[
            pl.BlockSpec((_T, _K), lambda i: (i, 0)),
            pl.BlockSpec((_T, _K), lambda i: (i, 0)),
            pl.BlockSpec(memory_space=pltpu.SMEM),
        ],
        out_shape=[
            jax.ShapeDtypeStruct((tokens, _K), jnp.float32),
            jax.ShapeDtypeStruct((tokens, _K), jnp.int32),
            jax.ShapeDtypeStruct((1, 1), jnp.float32),
        ],
        scratch_shapes=[pltpu.VMEM((1, _E), jnp.float32)],
        compiler_params=pltpu.CompilerParams(
            dimension_semantics=("arbitrary",),
        ),
    )(x2, x2, w, w)
    return (
        wts.reshape(_B, _N, _K),
        idx.reshape(_B, _N, _K),
        loss[0, 0],
    )


# speedup vs baseline: 1.5187x; 1.3505x over previous
import jax
import jax.numpy as jnp
from jax.experimental import pallas as pl
from jax.experimental.pallas import tpu as pltpu

_B, _N, _D = 4, 4096, 4096
_E = 64
_K = 8
_T = 1024


def _router_kernel(x_ref, w_ref, wts_ref, idx_ref, loss_ref, acc_ref):
    i = pl.program_id(0)
    nblocks = pl.num_programs(0)

    @pl.when(i == 0)
    def _init():
        acc_ref[...] = jnp.zeros_like(acc_ref)

    logits = jax.lax.dot_general(
        w_ref[...],
        x_ref[...],
        (((1,), (1,)), ((), ())),
        preferred_element_type=jnp.float32,
    )
    m = jnp.max(logits, axis=0, keepdims=True)
    p = jnp.exp(logits - m)
    s = jnp.sum(p, axis=0, keepdims=True)

    acc_ref[...] += p * (1.0 / s)

    iota = jax.lax.broadcasted_iota(jnp.int32, p.shape, 0)
    cur = ((jax.lax.bitcast_convert_type(p, jnp.int32) + 32) & ~(_E - 1)) | (
        (_E - 1) - iota
    )
    vals, idxs = [], []
    for _ in range(_K):
        mk = jnp.max(cur, axis=0, keepdims=True)
        idxs.append((_E - 1) - (mk & (_E - 1)))
        vals.append(jax.lax.bitcast_convert_type(mk & ~(_E - 1), jnp.float32))
        cur = jnp.where(cur == mk, jnp.int32(-(2**31)), cur)
    v = jnp.concatenate(vals, axis=0)
    wts_ref[...] = v * (1.0 / jnp.sum(v, axis=0, keepdims=True))
    idx_ref[...] = jnp.concatenate(idxs, axis=0)

    @pl.when(i == nblocks - 1)
    def _finish():
        usage = jnp.sum(acc_ref[...], axis=1, keepdims=True) / (nblocks * _T)
        loss_ref[0, 0] = jnp.sum(usage * jnp.log(usage * _E + 1e-8))


def kernel(x, gate_w):
    tokens = _B * _N
    x2 = x.reshape(tokens, _D)
    grid = tokens // _T
    wts, idx, loss = pl.pallas_call(
        _router_kernel,
        grid=(grid,),
        in_specs=[
            pl.BlockSpec((_T, _D), lambda i: (i, 0)),
            pl.BlockSpec((_E, _D), lambda i: (0, 0)),
        ],
        out_specs=[
            pl.BlockSpec((_K, _T), lambda i: (0, i)),
            pl.BlockSpec((_K, _T), lambda i: (0, i)),
            pl.BlockSpec(memory_space=pltpu.SMEM),
        ],
        out_shape=[
            jax.ShapeDtypeStruct((_K, tokens), jnp.float32),
            jax.ShapeDtypeStruct((_K, tokens), jnp.int32),
            jax.ShapeDtypeStruct((1, 1), jnp.float32),
        ],
        scratch_shapes=[pltpu.VMEM((_E, _T), jnp.float32)],
        compiler_params=pltpu.CompilerParams(
            dimension_semantics=("arbitrary",),
        ),
    )(x2, gate_w)
    return (
        wts.T.reshape(_B, _N, _K),
        idx.T.reshape(_B, _N, _K),
        loss[0, 0],
    )
